# baseline (speedup 1.0000x reference)
import functools

import jax
import jax.numpy as jnp
from jax import lax
from jax.experimental import pallas as pl
from jax.experimental.pallas import tpu as pltpu
from jax.experimental.pallas import tpu_sc as plsc

_CHUNKS = 4


def kernel(tokens, W_pos):
    batch, seq_len = tokens.shape
    n_ctx, d_model = W_pos.shape
    rows_per_chunk = seq_len // _CHUNKS

    tc_mesh = pltpu.create_tensorcore_mesh("t", num_cores=1)

    @functools.partial(
        pl.kernel,
        out_type=pltpu.HBM((batch, seq_len, d_model), W_pos.dtype),
        mesh=tc_mesh,
        scratch_types=[
            pltpu.VMEM((seq_len, d_model), W_pos.dtype),
            pltpu.SemaphoreType.DMA,
        ],
    )
    def body(w_hbm, out_hbm, buf, sem):
        pltpu.sync_copy(w_hbm, buf)
        copies = []
        for c in range(_CHUNKS):
            r0 = c * rows_per_chunk
            for b in range(batch):
                copies.append(
                    pltpu.async_copy(
                        buf.at[pl.ds(r0, rows_per_chunk), :],
                        out_hbm.at[b, pl.ds(r0, rows_per_chunk), :],
                        sem,
                    )
                )
        for c in copies:
            c.wait()

    return body(W_pos)
